# baseline (speedup 1.0000x reference)
import jax
import jax.numpy as jnp
from jax import lax
from jax.experimental import pallas as pl
from jax.experimental.pallas import tpu as pltpu
from jax.experimental.pallas import tpu_sc as plsc

ROWS = 128
N = 32768
K = 3276
NC, NS, L = 2, 16, 16
NW = NC * NS
RPW = ROWS // NW
VSTEPS = N // L
NB = 256
NG = NB // L

S0 = 0.9999999827814483
S1 = -0.16666651519624331
S2 = 0.008332964007287835
S3 = -0.0001980475458386001
S4 = 2.5981089066425313e-06
HALF_PI = 1.5707963267948966


def _body(phase_hbm, amp_hbm, out_hbm, act0_v, act1_v, am_v, hist_v,
          sem_p, sem_a, sem_o):
    wid = lax.axis_index("s") * NC + lax.axis_index("c")
    lane = lax.iota(jnp.int32, L)
    lane_base = lane * NB
    zeros_i = jnp.zeros((L,), jnp.int32)
    ones_i = jnp.ones((L,), jnp.int32)

    def _clear(b, _):
        hist_v[pl.ds(b * L, L)] = zeros_i
        return 0

    lax.fori_loop(0, NB * L // L, _clear, 0)

    def _select_threshold(kp):
        gsum = []
        tots = []
        sufs = []
        for g in range(NG):
            acc = hist_v[pl.ds(g * L, L)]
            hist_v[pl.ds(g * L, L)] = zeros_i
            for l in range(1, L):
                off = l * NB + g * L
                acc = acc + hist_v[pl.ds(off, L)]
                hist_v[pl.ds(off, L)] = zeros_i
            tots.append(acc)
            suf = lax.rev(plsc.cumsum(lax.rev(acc, (0,))), (0,))
            sufs.append(suf)
            gsum.append(jnp.sum(acc))
        big_g = jnp.int32(-1)
        car_g = jnp.int32(0)
        carry = jnp.int32(0)
        for g in range(NG - 1, -1, -1):
            hit = jnp.logical_and(big_g < 0, carry + gsum[g] >= kp)
            big_g = jnp.where(hit, g, big_g)
            car_g = jnp.where(hit, carry, car_g)
            carry = carry + gsum[g]
        suf_sel = zeros_i
        tot_sel = zeros_i
        for g in range(NG):
            cond = lax.broadcast_in_dim(big_g == g, (L,), ())
            suf_sel = jnp.where(cond, sufs[g], suf_sel)
            tot_sel = jnp.where(cond, tots[g], tot_sel)
        m = (suf_sel + car_g) >= kp
        binx = jnp.sum(m.astype(jnp.int32)) - 1
        byte = big_g * L + binx
        above = car_g + jnp.sum(jnp.where(lane > binx, tot_sel, zeros_i))
        return byte, kp - above

    def _process_row(r, act_v, nxt_v):
        row = wid * RPW + r

        @plsc.parallel_loop(0, VSTEPS, unroll=8)
        def _acth(i):
            for u in range(1):
                off = i * L
                ph = act_v[pl.ds(off, L)]
                am = am_v[pl.ds(off, L)]
                z = jnp.float32(HALF_PI) - ph * jnp.float32(0.5)
                z2 = z * z
                p = jnp.float32(S4)
                p = p * z2 + jnp.float32(S3)
                p = p * z2 + jnp.float32(S2)
                p = p * z2 + jnp.float32(S1)
                p = p * z2 + jnp.float32(S0)
                s = p * z
                a = am * (s * s)
                act_v[pl.ds(off, L)] = a
                key = lax.bitcast_convert_type(a, jnp.int32)
                idx = lane_base + lax.shift_right_logical(key, 24)
                plsc.addupdate_scatter(hist_v, [idx], ones_i)

        @pl.when(r < RPW - 1)
        def _prefetch():
            @pl.when(r >= 1)
            def _drain_out():
                pltpu.make_async_copy(
                    phase_hbm.at[row], nxt_v, sem_o
                ).wait()

            pltpu.make_async_copy(
                phase_hbm.at[row + 1], nxt_v, sem_p
            ).start()

        kp = jnp.int32(K)
        byte, kp = _select_threshold(kp)
        prefix = byte

        @plsc.parallel_loop(0, VSTEPS, unroll=8, carry=zeros_i)
        def _hist1(i, cnt):
            off = i * L
            key = lax.bitcast_convert_type(act_v[pl.ds(off, L)], jnp.int32)
            msk = lax.shift_right_logical(key, 24) == prefix
            bucket = lax.shift_right_logical(key, 16) & 255
            plsc.addupdate_scatter(hist_v, [lane_base + bucket], ones_i,
                                   mask=msk)
            plsc.store_scatter(am_v, [cnt * L + lane],
                               lax.bitcast_convert_type(key, jnp.float32),
                               mask=msk)
            return cnt + msk.astype(jnp.int32)

        cnt_vec = _hist1
        n_max = jnp.max(cnt_vec)

        byte, kp = _select_threshold(kp)
        prefix = (prefix << 8) | byte

        for shift in (8, 0):

            @plsc.parallel_loop(0, n_max, unroll=4)
            def _hist2(i, shift=shift, prefix=prefix):
                key = lax.bitcast_convert_type(am_v[pl.ds(i * L, L)],
                                               jnp.int32)
                msk = jnp.logical_and(
                    i < cnt_vec,
                    lax.shift_right_logical(key, shift + 8) == prefix,
                )
                bucket = lax.shift_right_logical(key, shift) & 255
                plsc.addupdate_scatter(hist_v, [lane_base + bucket], ones_i,
                                       mask=msk)

            if shift == 0:
                @pl.when(r < RPW - 1)
                def _prefetch_am():
                    pltpu.make_async_copy(
                        amp_hbm.at[row + 1], am_v, sem_a
                    ).start()

            byte, kp = _select_threshold(kp)
            prefix = (prefix << 8) | byte

        thresh_bits = prefix

        @plsc.parallel_loop(0, VSTEPS // 8, unroll=2)
        def _mask(i):
            for u in range(8):
                off = (i * 8 + u) * L
                a = act_v[pl.ds(off, L)]
                keep = lax.bitcast_convert_type(a, jnp.int32) >= thresh_bits
                act_v[pl.ds(off, L)] = jnp.where(keep, a, jnp.float32(0.0))

        pltpu.make_async_copy(act_v, out_hbm.at[row], sem_o).start()

        @pl.when(r < RPW - 1)
        def _wait_in():
            pltpu.make_async_copy(
                phase_hbm.at[row + 1], nxt_v, sem_p
            ).wait()
            pltpu.make_async_copy(amp_hbm.at[row + 1], am_v, sem_a).wait()

    row0 = wid * RPW
    cp_p = pltpu.make_async_copy(phase_hbm.at[row0], act0_v, sem_p)
    cp_a = pltpu.make_async_copy(amp_hbm.at[row0], am_v, sem_a)
    cp_p.start()
    cp_a.start()
    cp_p.wait()
    cp_a.wait()

    def _row_pair(rp, _):
        _process_row(rp * 2, act0_v, act1_v)
        _process_row(rp * 2 + 1, act1_v, act0_v)
        return 0

    lax.fori_loop(0, RPW // 2, _row_pair, 0)

    pltpu.make_async_copy(act0_v, out_hbm.at[row0], sem_o).wait()
    pltpu.make_async_copy(act1_v, out_hbm.at[row0], sem_o).wait()


def kernel(phase, amplitude, temperature):
    del temperature
    mesh = plsc.VectorSubcoreMesh(core_axis_name="c", subcore_axis_name="s")
    run = pl.kernel(
        _body,
        out_type=jax.ShapeDtypeStruct((ROWS, N), jnp.float32),
        mesh=mesh,
        compiler_params=pltpu.CompilerParams(needs_layout_passes=False),
        scratch_types=[
            pltpu.VMEM((N,), jnp.float32),
            pltpu.VMEM((N,), jnp.float32),
            pltpu.VMEM((N,), jnp.float32),
            pltpu.VMEM((NB * L,), jnp.int32),
            pltpu.SemaphoreType.DMA,
            pltpu.SemaphoreType.DMA,
            pltpu.SemaphoreType.DMA,
        ],
    )
    return run(phase, amplitude)


# speedup vs baseline: 1.1315x; 1.1315x over previous
import jax
import jax.numpy as jnp
from jax import lax
from jax.experimental import pallas as pl
from jax.experimental.pallas import tpu as pltpu
from jax.experimental.pallas import tpu_sc as plsc

ROWS = 128
N = 32768
K = 3276
NC, NS, L = 2, 16, 16
NW = NC * NS
RPW = ROWS // NW
VSTEPS = N // L
NB = 256
NG = NB // L

S0 = 0.9999999827814483
S1 = -0.16666651519624331
S2 = 0.008332964007287835
S3 = -0.0001980475458386001
S4 = 2.5981089066425313e-06
HALF_PI = 1.5707963267948966


def _body(phase_hbm, amp_hbm, out_hbm, act0_v, act1_v, am_v, hist_v,
          sem_p, sem_a, sem_o):
    wid = lax.axis_index("s") * NC + lax.axis_index("c")
    lane = lax.iota(jnp.int32, L)
    zeros_i = jnp.zeros((L,), jnp.int32)
    ones_i = jnp.ones((L,), jnp.int32)

    def _clear(b, _):
        hist_v[pl.ds(b * L, L)] = zeros_i
        return 0

    lax.fori_loop(0, NB * L // L, _clear, 0)

    def _select_threshold(kp):
        gsum = []
        for g in range(NG):
            acc = hist_v[pl.ds(g * NB, L)]
            for j in range(1, L):
                acc = acc + hist_v[pl.ds(g * NB + j * L, L)]
            gsum.append(jnp.sum(acc))
        big_g = jnp.int32(-1)
        car_g = jnp.int32(0)
        carry = jnp.int32(0)
        for g in range(NG - 1, -1, -1):
            hit = jnp.logical_and(big_g < 0, carry + gsum[g] >= kp)
            big_g = jnp.where(hit, g, big_g)
            car_g = jnp.where(hit, carry, car_g)
            carry = carry + gsum[g]
        base = big_g * NB
        tj = [jnp.sum(hist_v[pl.ds(base + j * L, L)]) for j in range(L)]
        cum = car_g
        binx = jnp.int32(-1)
        above = jnp.int32(0)
        for j in range(L - 1, -1, -1):
            hit = jnp.logical_and(binx < 0, cum + tj[j] >= kp)
            binx = jnp.where(hit, j, binx)
            above = jnp.where(hit, cum, above)
            cum = cum + tj[j]
        byte = big_g * L + binx

        @plsc.parallel_loop(0, NB * L // (8 * L))
        def _clear8(i):
            for u in range(8):
                hist_v[pl.ds((i * 8 + u) * L, L)] = zeros_i

        return byte, kp - above

    def _process_row(r, act_v, nxt_v):
        row = wid * RPW + r

        @plsc.parallel_loop(0, VSTEPS, unroll=8)
        def _acth(i):
            for u in range(1):
                off = i * L
                ph = act_v[pl.ds(off, L)]
                am = am_v[pl.ds(off, L)]
                z = jnp.float32(HALF_PI) - ph * jnp.float32(0.5)
                z2 = z * z
                p = jnp.float32(S4)
                p = p * z2 + jnp.float32(S3)
                p = p * z2 + jnp.float32(S2)
                p = p * z2 + jnp.float32(S1)
                p = p * z2 + jnp.float32(S0)
                s = p * z
                a = am * (s * s)
                act_v[pl.ds(off, L)] = a
                key = lax.bitcast_convert_type(a, jnp.int32)
                idx = lax.shift_right_logical(key, 24) * L + lane
                plsc.addupdate_scatter(hist_v, [idx], ones_i)

        @pl.when(r < RPW - 1)
        def _prefetch():
            @pl.when(r >= 1)
            def _drain_out():
                pltpu.make_async_copy(
                    phase_hbm.at[row], nxt_v, sem_o
                ).wait()

            pltpu.make_async_copy(
                phase_hbm.at[row + 1], nxt_v, sem_p
            ).start()

        kp = jnp.int32(K)
        byte, kp = _select_threshold(kp)
        prefix = byte

        @plsc.parallel_loop(0, VSTEPS, unroll=8, carry=zeros_i)
        def _hist1(i, cnt):
            off = i * L
            key = lax.bitcast_convert_type(act_v[pl.ds(off, L)], jnp.int32)
            msk = lax.shift_right_logical(key, 24) == prefix
            bucket = lax.shift_right_logical(key, 16) & 255
            plsc.addupdate_scatter(hist_v, [bucket * L + lane], ones_i,
                                   mask=msk)
            plsc.store_scatter(am_v, [cnt * L + lane],
                               lax.bitcast_convert_type(key, jnp.float32),
                               mask=msk)
            return cnt + msk.astype(jnp.int32)

        cnt_vec = _hist1
        n_max = jnp.max(cnt_vec)

        byte, kp = _select_threshold(kp)
        prefix = (prefix << 8) | byte

        for shift in (8, 0):

            @plsc.parallel_loop(0, n_max, unroll=4)
            def _hist2(i, shift=shift, prefix=prefix):
                key = lax.bitcast_convert_type(am_v[pl.ds(i * L, L)],
                                               jnp.int32)
                msk = jnp.logical_and(
                    i < cnt_vec,
                    lax.shift_right_logical(key, shift + 8) == prefix,
                )
                bucket = lax.shift_right_logical(key, shift) & 255
                plsc.addupdate_scatter(hist_v, [bucket * L + lane], ones_i,
                                       mask=msk)

            if shift == 0:
                @pl.when(r < RPW - 1)
                def _prefetch_am():
                    pltpu.make_async_copy(
                        amp_hbm.at[row + 1], am_v, sem_a
                    ).start()

            byte, kp = _select_threshold(kp)
            prefix = (prefix << 8) | byte

        thresh_bits = prefix

        @plsc.parallel_loop(0, VSTEPS // 8, unroll=2)
        def _mask(i):
            for u in range(8):
                off = (i * 8 + u) * L
                a = act_v[pl.ds(off, L)]
                keep = lax.bitcast_convert_type(a, jnp.int32) >= thresh_bits
                act_v[pl.ds(off, L)] = jnp.where(keep, a, jnp.float32(0.0))

        pltpu.make_async_copy(act_v, out_hbm.at[row], sem_o).start()

        @pl.when(r < RPW - 1)
        def _wait_in():
            pltpu.make_async_copy(
                phase_hbm.at[row + 1], nxt_v, sem_p
            ).wait()
            pltpu.make_async_copy(amp_hbm.at[row + 1], am_v, sem_a).wait()

    row0 = wid * RPW
    cp_p = pltpu.make_async_copy(phase_hbm.at[row0], act0_v, sem_p)
    cp_a = pltpu.make_async_copy(amp_hbm.at[row0], am_v, sem_a)
    cp_p.start()
    cp_a.start()
    cp_p.wait()
    cp_a.wait()

    def _row_pair(rp, _):
        _process_row(rp * 2, act0_v, act1_v)
        _process_row(rp * 2 + 1, act1_v, act0_v)
        return 0

    lax.fori_loop(0, RPW // 2, _row_pair, 0)

    pltpu.make_async_copy(act0_v, out_hbm.at[row0], sem_o).wait()
    pltpu.make_async_copy(act1_v, out_hbm.at[row0], sem_o).wait()


def kernel(phase, amplitude, temperature):
    del temperature
    mesh = plsc.VectorSubcoreMesh(core_axis_name="c", subcore_axis_name="s")
    run = pl.kernel(
        _body,
        out_type=jax.ShapeDtypeStruct((ROWS, N), jnp.float32),
        mesh=mesh,
        compiler_params=pltpu.CompilerParams(needs_layout_passes=False),
        scratch_types=[
            pltpu.VMEM((N,), jnp.float32),
            pltpu.VMEM((N,), jnp.float32),
            pltpu.VMEM((N,), jnp.float32),
            pltpu.VMEM((NB * L,), jnp.int32),
            pltpu.SemaphoreType.DMA,
            pltpu.SemaphoreType.DMA,
            pltpu.SemaphoreType.DMA,
        ],
    )
    return run(phase, amplitude)


# speedup vs baseline: 1.1548x; 1.0206x over previous
import jax
import jax.numpy as jnp
from jax import lax
from jax.experimental import pallas as pl
from jax.experimental.pallas import tpu as pltpu
from jax.experimental.pallas import tpu_sc as plsc

ROWS = 128
N = 32768
K = 3276
NC, NS, L = 2, 16, 16
NW = NC * NS
RPW = ROWS // NW
VSTEPS = N // L
NB = 256
NG = NB // L

S0 = 0.9999999827814483
S1 = -0.16666651519624331
S2 = 0.008332964007287835
S3 = -0.0001980475458386001
S4 = 2.5981089066425313e-06
C0 = S0 / 2.0
C1 = S1 / 8.0
C2 = S2 / 32.0
C3 = S3 / 128.0
C4 = S4 / 512.0
PI = 3.141592653589793


def _body(phase_hbm, amp_hbm, out_hbm, act0_v, act1_v, am_v, hist_v,
          sem_p, sem_a, sem_o):
    wid = lax.axis_index("s") * NC + lax.axis_index("c")
    lane = lax.iota(jnp.int32, L)
    zeros_i = jnp.zeros((L,), jnp.int32)
    ones_i = jnp.ones((L,), jnp.int32)

    def _clear(b, _):
        hist_v[pl.ds(b * L, L)] = zeros_i
        return 0

    lax.fori_loop(0, NB * L // L, _clear, 0)

    def _select_threshold(kp):
        gsum = []
        for g in range(NG):
            acc = hist_v[pl.ds(g * NB, L)]
            for j in range(1, L):
                acc = acc + hist_v[pl.ds(g * NB + j * L, L)]
            gsum.append(jnp.sum(acc))
        big_g = jnp.int32(-1)
        car_g = jnp.int32(0)
        carry = jnp.int32(0)
        for g in range(NG - 1, -1, -1):
            hit = jnp.logical_and(big_g < 0, carry + gsum[g] >= kp)
            big_g = jnp.where(hit, g, big_g)
            car_g = jnp.where(hit, carry, car_g)
            carry = carry + gsum[g]
        base = big_g * NB
        tj = [jnp.sum(hist_v[pl.ds(base + j * L, L)]) for j in range(L)]
        cum = car_g
        binx = jnp.int32(-1)
        above = jnp.int32(0)
        for j in range(L - 1, -1, -1):
            hit = jnp.logical_and(binx < 0, cum + tj[j] >= kp)
            binx = jnp.where(hit, j, binx)
            above = jnp.where(hit, cum, above)
            cum = cum + tj[j]
        byte = big_g * L + binx

        @plsc.parallel_loop(0, NB * L // (8 * L))
        def _clear8(i):
            for u in range(8):
                hist_v[pl.ds((i * 8 + u) * L, L)] = zeros_i

        return byte, kp - above

    def _process_row(r, act_v, nxt_v):
        row = wid * RPW + r

        @plsc.parallel_loop(0, VSTEPS, unroll=8)
        def _acth(i):
            off = i * L
            ph = act_v[pl.ds(off, L)]
            am = am_v[pl.ds(off, L)]
            z = jnp.float32(PI) - ph
            z2 = z * z
            p = jnp.float32(C4)
            p = p * z2 + jnp.float32(C3)
            p = p * z2 + jnp.float32(C2)
            p = p * z2 + jnp.float32(C1)
            p = p * z2 + jnp.float32(C0)
            s = p * z
            a = am * (s * s)
            act_v[pl.ds(off, L)] = a
            key = lax.bitcast_convert_type(a, jnp.int32)
            idx = lax.shift_right_logical(key, 24) * L + lane
            plsc.addupdate_scatter(hist_v, [idx], ones_i)

        @pl.when(r < RPW - 1)
        def _prefetch():
            @pl.when(r >= 1)
            def _drain_out():
                pltpu.make_async_copy(
                    phase_hbm.at[row], nxt_v, sem_o
                ).wait()

            pltpu.make_async_copy(
                phase_hbm.at[row + 1], nxt_v, sem_p
            ).start()

        kp = jnp.int32(K)
        byte, kp = _select_threshold(kp)
        prefix = byte

        @plsc.parallel_loop(0, VSTEPS, unroll=8, carry=zeros_i)
        def _hist1(i, cnt):
            off = i * L
            key = lax.bitcast_convert_type(act_v[pl.ds(off, L)], jnp.int32)
            msk = lax.shift_right_logical(key, 24) == prefix
            bucket = lax.shift_right_logical(key, 16) & 255
            plsc.addupdate_scatter(hist_v, [bucket * L + lane], ones_i,
                                   mask=msk)
            plsc.store_scatter(am_v, [cnt * L + lane],
                               lax.bitcast_convert_type(key, jnp.float32),
                               mask=msk)
            return cnt + msk.astype(jnp.int32)

        cnt_vec = _hist1
        n_max = jnp.max(cnt_vec)

        byte, kp = _select_threshold(kp)
        prefix = (prefix << 8) | byte

        for shift in (8, 0):

            @plsc.parallel_loop(0, n_max, unroll=4)
            def _hist2(i, shift=shift, prefix=prefix):
                key = lax.bitcast_convert_type(am_v[pl.ds(i * L, L)],
                                               jnp.int32)
                msk = jnp.logical_and(
                    i < cnt_vec,
                    lax.shift_right_logical(key, shift + 8) == prefix,
                )
                bucket = lax.shift_right_logical(key, shift) & 255
                plsc.addupdate_scatter(hist_v, [bucket * L + lane], ones_i,
                                       mask=msk)

            if shift == 0:
                @pl.when(r < RPW - 1)
                def _prefetch_am():
                    pltpu.make_async_copy(
                        amp_hbm.at[row + 1], am_v, sem_a
                    ).start()

            byte, kp = _select_threshold(kp)
            prefix = (prefix << 8) | byte

        thresh_bits = prefix

        @plsc.parallel_loop(0, VSTEPS // 8, unroll=2)
        def _mask(i):
            for u in range(8):
                off = (i * 8 + u) * L
                a = act_v[pl.ds(off, L)]
                keep = lax.bitcast_convert_type(a, jnp.int32) >= thresh_bits
                act_v[pl.ds(off, L)] = jnp.where(keep, a, jnp.float32(0.0))

        pltpu.make_async_copy(act_v, out_hbm.at[row], sem_o).start()

        @pl.when(r < RPW - 1)
        def _wait_in():
            pltpu.make_async_copy(
                phase_hbm.at[row + 1], nxt_v, sem_p
            ).wait()
            pltpu.make_async_copy(amp_hbm.at[row + 1], am_v, sem_a).wait()

    row0 = wid * RPW
    cp_p = pltpu.make_async_copy(phase_hbm.at[row0], act0_v, sem_p)
    cp_a = pltpu.make_async_copy(amp_hbm.at[row0], am_v, sem_a)
    cp_p.start()
    cp_a.start()
    cp_p.wait()
    cp_a.wait()

    def _row_pair(rp, _):
        _process_row(rp * 2, act0_v, act1_v)
        _process_row(rp * 2 + 1, act1_v, act0_v)
        return 0

    lax.fori_loop(0, RPW // 2, _row_pair, 0)

    pltpu.make_async_copy(act0_v, out_hbm.at[row0], sem_o).wait()
    pltpu.make_async_copy(act1_v, out_hbm.at[row0], sem_o).wait()


def kernel(phase, amplitude, temperature):
    del temperature
    mesh = plsc.VectorSubcoreMesh(core_axis_name="c", subcore_axis_name="s")
    run = pl.kernel(
        _body,
        out_type=jax.ShapeDtypeStruct((ROWS, N), jnp.float32),
        mesh=mesh,
        compiler_params=pltpu.CompilerParams(needs_layout_passes=False),
        scratch_types=[
            pltpu.VMEM((N,), jnp.float32),
            pltpu.VMEM((N,), jnp.float32),
            pltpu.VMEM((N,), jnp.float32),
            pltpu.VMEM((NB * L,), jnp.int32),
            pltpu.SemaphoreType.DMA,
            pltpu.SemaphoreType.DMA,
            pltpu.SemaphoreType.DMA,
        ],
    )
    return run(phase, amplitude)
